# baseline (speedup 1.0000x reference)
import functools

import jax
import jax.numpy as jnp
from jax import lax
from jax.experimental import pallas as pl
from jax.experimental.pallas import tpu as pltpu
from jax.experimental.pallas import tpu_sc as plsc

_LANES = 16


def _sc_gather_coeffs(table_a, table_b, timesteps):
    num_steps = table_a.shape[0]
    batch = timesteps.shape[0]
    mesh = plsc.VectorSubcoreMesh(core_axis_name="c", subcore_axis_name="s")
    num_workers = mesh.num_cores * mesh.num_subcores
    bpw = batch // num_workers

    @functools.partial(
        pl.kernel,
        out_type=(
            jax.ShapeDtypeStruct((batch,), jnp.float32),
            jax.ShapeDtypeStruct((batch,), jnp.float32),
        ),
        mesh=mesh,
        compiler_params=pltpu.CompilerParams(needs_layout_passes=False),
        scratch_types=[
            pltpu.VMEM((bpw,), jnp.int32),
            pltpu.VMEM((num_steps,), jnp.float32),
            pltpu.VMEM((num_steps,), jnp.float32),
            pltpu.VMEM((bpw,), jnp.float32),
            pltpu.VMEM((bpw,), jnp.float32),
        ],
    )
    def gather_kernel(ta_hbm, tb_hbm, ts_hbm, a_hbm, b_hbm,
                      idx_v, ta_v, tb_v, av_v, bv_v):
        wid = lax.axis_index("s") * mesh.num_cores + lax.axis_index("c")
        base = wid * bpw
        pltpu.sync_copy(ts_hbm.at[pl.ds(base, bpw)], idx_v)
        pltpu.sync_copy(ta_hbm, ta_v)
        pltpu.sync_copy(tb_hbm, tb_v)

        @plsc.parallel_loop(0, bpw, _LANES, unroll=4)
        def gather_body(off):
            iv = idx_v[pl.ds(off, _LANES)]
            av_v[pl.ds(off, _LANES)] = plsc.load_gather(ta_v, [iv])
            bv_v[pl.ds(off, _LANES)] = plsc.load_gather(tb_v, [iv])

        pltpu.sync_copy(av_v, a_hbm.at[pl.ds(base, bpw)])
        pltpu.sync_copy(bv_v, b_hbm.at[pl.ds(base, bpw)])

    return gather_kernel(table_a, table_b, timesteps)


def _tc_scale_add(x_start, noise, coeff_a, coeff_b):
    batch, dim = x_start.shape
    blk = 4096
    grid = (batch // blk,)

    def body(x_ref, n_ref, a_ref, b_ref, o_ref):
        a = a_ref[...][:, None]
        b = b_ref[...][:, None]
        o_ref[...] = a * x_ref[...] + b * n_ref[...]

    return pl.pallas_call(
        body,
        grid=grid,
        in_specs=[
            pl.BlockSpec((blk, dim), lambda i: (i, 0)),
            pl.BlockSpec((blk, dim), lambda i: (i, 0)),
            pl.BlockSpec((blk,), lambda i: (i,)),
            pl.BlockSpec((blk,), lambda i: (i,)),
        ],
        out_specs=pl.BlockSpec((blk, dim), lambda i: (i, 0)),
        out_shape=jax.ShapeDtypeStruct((batch, dim), jnp.float32),
        compiler_params=pltpu.CompilerParams(
            dimension_semantics=("arbitrary",)),
    )(x_start, noise, coeff_a, coeff_b)


def kernel(x_start, noise, sqrt_alphas_cumprod, sqrt_one_minus_alphas_cumprod,
           timesteps):
    ts = timesteps.astype(jnp.int32)
    coeff_a, coeff_b = _sc_gather_coeffs(
        sqrt_alphas_cumprod, sqrt_one_minus_alphas_cumprod, ts)
    return _tc_scale_add(x_start, noise, coeff_a, coeff_b)


# speedup vs baseline: 1.0338x; 1.0338x over previous
import functools

import jax
import jax.numpy as jnp
from jax import lax
from jax.experimental import pallas as pl
from jax.experimental.pallas import tpu as pltpu
from jax.experimental.pallas import tpu_sc as plsc

_LANES = 16


def _sc_gather_coeffs(table_a, table_b, timesteps):
    num_steps = table_a.shape[0]
    batch = timesteps.shape[0]
    mesh = plsc.VectorSubcoreMesh(core_axis_name="c", subcore_axis_name="s")
    num_workers = mesh.num_cores * mesh.num_subcores
    bpw = batch // num_workers

    @functools.partial(
        pl.kernel,
        out_type=(
            jax.ShapeDtypeStruct((batch,), jnp.float32),
            jax.ShapeDtypeStruct((batch,), jnp.float32),
        ),
        mesh=mesh,
        compiler_params=pltpu.CompilerParams(needs_layout_passes=False),
        scratch_types=[
            pltpu.VMEM((bpw,), jnp.int32),
            pltpu.VMEM((num_steps,), jnp.float32),
            pltpu.VMEM((num_steps,), jnp.float32),
            pltpu.VMEM((bpw,), jnp.float32),
            pltpu.VMEM((bpw,), jnp.float32),
            pltpu.SemaphoreType.DMA,
        ],
    )
    def gather_kernel(ta_hbm, tb_hbm, ts_hbm, a_hbm, b_hbm,
                      idx_v, ta_v, tb_v, av_v, bv_v, sem):
        wid = lax.axis_index("s") * mesh.num_cores + lax.axis_index("c")
        base = wid * bpw
        copies = (
            pltpu.make_async_copy(ts_hbm.at[pl.ds(base, bpw)], idx_v, sem),
            pltpu.make_async_copy(ta_hbm, ta_v, sem),
            pltpu.make_async_copy(tb_hbm, tb_v, sem),
        )
        for cp in copies:
            cp.start()
        for cp in copies:
            cp.wait()

        @plsc.parallel_loop(0, bpw, _LANES, unroll=4)
        def gather_body(off):
            iv = idx_v[pl.ds(off, _LANES)]
            av_v[pl.ds(off, _LANES)] = plsc.load_gather(ta_v, [iv])
            bv_v[pl.ds(off, _LANES)] = plsc.load_gather(tb_v, [iv])

        pltpu.sync_copy(av_v, a_hbm.at[pl.ds(base, bpw)])
        pltpu.sync_copy(bv_v, b_hbm.at[pl.ds(base, bpw)])

    return gather_kernel(table_a, table_b, timesteps)


def _tc_scale_add(x_start, noise, coeff_a, coeff_b):
    batch, dim = x_start.shape
    blk = 4096
    grid = (batch // blk,)

    def body(x_ref, n_ref, a_ref, b_ref, o_ref):
        a = a_ref[...][:, None]
        b = b_ref[...][:, None]
        o_ref[...] = a * x_ref[...] + b * n_ref[...]

    return pl.pallas_call(
        body,
        grid=grid,
        in_specs=[
            pl.BlockSpec((blk, dim), lambda i: (i, 0)),
            pl.BlockSpec((blk, dim), lambda i: (i, 0)),
            pl.BlockSpec((blk,), lambda i: (i,)),
            pl.BlockSpec((blk,), lambda i: (i,)),
        ],
        out_specs=pl.BlockSpec((blk, dim), lambda i: (i, 0)),
        out_shape=jax.ShapeDtypeStruct((batch, dim), jnp.float32),
        compiler_params=pltpu.CompilerParams(
            dimension_semantics=("arbitrary",)),
    )(x_start, noise, coeff_a, coeff_b)


def kernel(x_start, noise, sqrt_alphas_cumprod, sqrt_one_minus_alphas_cumprod,
           timesteps):
    ts = timesteps.astype(jnp.int32)
    coeff_a, coeff_b = _sc_gather_coeffs(
        sqrt_alphas_cumprod, sqrt_one_minus_alphas_cumprod, ts)
    return _tc_scale_add(x_start, noise, coeff_a, coeff_b)


# speedup vs baseline: 1.0645x; 1.0297x over previous
import functools

import jax
import jax.numpy as jnp
from jax import lax
from jax.experimental import pallas as pl
from jax.experimental.pallas import tpu as pltpu
from jax.experimental.pallas import tpu_sc as plsc

_LANES = 16


def _sc_gather_coeffs(table_a, table_b, timesteps):
    num_steps = table_a.shape[0]
    batch = timesteps.shape[0]
    mesh = plsc.VectorSubcoreMesh(core_axis_name="c", subcore_axis_name="s")
    num_workers = mesh.num_cores * mesh.num_subcores
    bpw = batch // num_workers

    @functools.partial(
        pl.kernel,
        out_type=(
            jax.ShapeDtypeStruct((batch,), jnp.float32),
            jax.ShapeDtypeStruct((batch,), jnp.float32),
        ),
        mesh=mesh,
        compiler_params=pltpu.CompilerParams(needs_layout_passes=False),
        scratch_types=[
            pltpu.VMEM((bpw,), jnp.int32),
            pltpu.VMEM((num_steps,), jnp.float32),
            pltpu.VMEM((num_steps,), jnp.float32),
            pltpu.VMEM((bpw,), jnp.float32),
            pltpu.VMEM((bpw,), jnp.float32),
            pltpu.SemaphoreType.DMA,
        ],
    )
    def gather_kernel(ta_hbm, tb_hbm, ts_hbm, a_hbm, b_hbm,
                      idx_v, ta_v, tb_v, av_v, bv_v, sem):
        wid = lax.axis_index("s") * mesh.num_cores + lax.axis_index("c")
        base = wid * bpw
        copies = (
            pltpu.make_async_copy(ts_hbm.at[pl.ds(base, bpw)], idx_v, sem),
            pltpu.make_async_copy(ta_hbm, ta_v, sem),
            pltpu.make_async_copy(tb_hbm, tb_v, sem),
        )
        for cp in copies:
            cp.start()
        for cp in copies:
            cp.wait()

        @plsc.parallel_loop(0, bpw, _LANES, unroll=4)
        def gather_body(off):
            iv = idx_v[pl.ds(off, _LANES)]
            av_v[pl.ds(off, _LANES)] = plsc.load_gather(ta_v, [iv])
            bv_v[pl.ds(off, _LANES)] = plsc.load_gather(tb_v, [iv])

        pltpu.sync_copy(av_v, a_hbm.at[pl.ds(base, bpw)])
        pltpu.sync_copy(bv_v, b_hbm.at[pl.ds(base, bpw)])

    return gather_kernel(table_a, table_b, timesteps)


def _tc_scale_add(x_start, noise, coeff_a, coeff_b):
    batch, dim = x_start.shape
    blk = 8192
    grid = (batch // blk,)

    def body(x_ref, n_ref, a_ref, b_ref, o_ref):
        a = a_ref[...][:, None]
        b = b_ref[...][:, None]
        o_ref[...] = a * x_ref[...] + b * n_ref[...]

    return pl.pallas_call(
        body,
        grid=grid,
        in_specs=[
            pl.BlockSpec((blk, dim), lambda i: (i, 0)),
            pl.BlockSpec((blk, dim), lambda i: (i, 0)),
            pl.BlockSpec((blk,), lambda i: (i,)),
            pl.BlockSpec((blk,), lambda i: (i,)),
        ],
        out_specs=pl.BlockSpec((blk, dim), lambda i: (i, 0)),
        out_shape=jax.ShapeDtypeStruct((batch, dim), jnp.float32),
        compiler_params=pltpu.CompilerParams(
            dimension_semantics=("arbitrary",)),
    )(x_start, noise, coeff_a, coeff_b)


def kernel(x_start, noise, sqrt_alphas_cumprod, sqrt_one_minus_alphas_cumprod,
           timesteps):
    ts = timesteps.astype(jnp.int32)
    coeff_a, coeff_b = _sc_gather_coeffs(
        sqrt_alphas_cumprod, sqrt_one_minus_alphas_cumprod, ts)
    return _tc_scale_add(x_start, noise, coeff_a, coeff_b)


# speedup vs baseline: 1.0647x; 1.0002x over previous
import functools

import jax
import jax.numpy as jnp
from jax import lax
from jax.experimental import pallas as pl
from jax.experimental.pallas import tpu as pltpu
from jax.experimental.pallas import tpu_sc as plsc

_LANES = 16


def _sc_gather_coeffs(table_a, table_b, timesteps):
    num_steps = table_a.shape[0]
    batch = timesteps.shape[0]
    mesh = plsc.VectorSubcoreMesh(core_axis_name="c", subcore_axis_name="s")
    num_workers = mesh.num_cores * mesh.num_subcores
    bpw = batch // num_workers

    @functools.partial(
        pl.kernel,
        out_type=(
            jax.ShapeDtypeStruct((batch,), jnp.float32),
            jax.ShapeDtypeStruct((batch,), jnp.float32),
        ),
        mesh=mesh,
        compiler_params=pltpu.CompilerParams(
            needs_layout_passes=False, skip_device_barrier=True),
        scratch_types=[
            pltpu.VMEM((bpw,), jnp.int32),
            pltpu.VMEM((num_steps,), jnp.float32),
            pltpu.VMEM((num_steps,), jnp.float32),
            pltpu.VMEM((bpw,), jnp.float32),
            pltpu.VMEM((bpw,), jnp.float32),
            pltpu.SemaphoreType.DMA,
        ],
    )
    def gather_kernel(ta_hbm, tb_hbm, ts_hbm, a_hbm, b_hbm,
                      idx_v, ta_v, tb_v, av_v, bv_v, sem):
        wid = lax.axis_index("s") * mesh.num_cores + lax.axis_index("c")
        base = wid * bpw
        copies = (
            pltpu.make_async_copy(ts_hbm.at[pl.ds(base, bpw)], idx_v, sem),
            pltpu.make_async_copy(ta_hbm, ta_v, sem),
            pltpu.make_async_copy(tb_hbm, tb_v, sem),
        )
        for cp in copies:
            cp.start()
        for cp in copies:
            cp.wait()

        @plsc.parallel_loop(0, bpw, _LANES, unroll=4)
        def gather_body(off):
            iv = idx_v[pl.ds(off, _LANES)]
            av_v[pl.ds(off, _LANES)] = plsc.load_gather(ta_v, [iv])
            bv_v[pl.ds(off, _LANES)] = plsc.load_gather(tb_v, [iv])

        pltpu.sync_copy(av_v, a_hbm.at[pl.ds(base, bpw)])
        pltpu.sync_copy(bv_v, b_hbm.at[pl.ds(base, bpw)])

    return gather_kernel(table_a, table_b, timesteps)


def _tc_scale_add(x_start, noise, coeff_a, coeff_b):
    batch, dim = x_start.shape
    blk = 8192
    grid = (batch // blk,)

    def body(x_ref, n_ref, a_ref, b_ref, o_ref):
        a = a_ref[...][:, None]
        b = b_ref[...][:, None]
        o_ref[...] = a * x_ref[...] + b * n_ref[...]

    return pl.pallas_call(
        body,
        grid=grid,
        in_specs=[
            pl.BlockSpec((blk, dim), lambda i: (i, 0)),
            pl.BlockSpec((blk, dim), lambda i: (i, 0)),
            pl.BlockSpec((blk,), lambda i: (i,)),
            pl.BlockSpec((blk,), lambda i: (i,)),
        ],
        out_specs=pl.BlockSpec((blk, dim), lambda i: (i, 0)),
        out_shape=jax.ShapeDtypeStruct((batch, dim), jnp.float32),
        compiler_params=pltpu.CompilerParams(
            dimension_semantics=("arbitrary",)),
    )(x_start, noise, coeff_a, coeff_b)


def kernel(x_start, noise, sqrt_alphas_cumprod, sqrt_one_minus_alphas_cumprod,
           timesteps):
    ts = timesteps.astype(jnp.int32)
    coeff_a, coeff_b = _sc_gather_coeffs(
        sqrt_alphas_cumprod, sqrt_one_minus_alphas_cumprod, ts)
    return _tc_scale_add(x_start, noise, coeff_a, coeff_b)
